# baseline (speedup 1.0000x reference)
import functools

import jax
import jax.numpy as jnp
from jax import lax
from jax.experimental import pallas as pl
from jax.experimental.pallas import tpu as pltpu
from jax.experimental.pallas import tpu_sc as plsc

NC = 2
NS = 16
NW = NC * NS
CHUNK = 128


def _sc_gather(table, idx3):
    _, chunks, chunk = idx3.shape
    d = table.shape[1]
    per_w = chunks * chunk
    mesh = plsc.VectorSubcoreMesh(core_axis_name="c", subcore_axis_name="s")

    @functools.partial(
        pl.kernel,
        mesh=mesh,
        compiler_params=pltpu.CompilerParams(use_tc_tiling_on_sc=False),
        out_type=jax.ShapeDtypeStruct((NW * per_w, d), jnp.float32),
        scratch_types=[
            pltpu.VMEM((chunks, chunk), jnp.int32),
            pltpu.VMEM((4, chunk, d), jnp.float32),
            pltpu.SemaphoreType.DMA,
            pltpu.SemaphoreType.DMA,
            pltpu.SemaphoreType.DMA,
            pltpu.SemaphoreType.DMA,
            pltpu.SemaphoreType.DMA,
            pltpu.SemaphoreType.DMA,
            pltpu.SemaphoreType.DMA,
            pltpu.SemaphoreType.DMA,
        ],
    )
    def k(table_hbm, idx_hbm, out_hbm, idx_v, rows_v,
          g0, g1, g2, g3, o0, o1, o2, o3):
        wid = lax.axis_index("s") * NC + lax.axis_index("c")
        pltpu.sync_copy(idx_hbm.at[wid], idx_v)
        base = wid * per_w
        gsems = [g0, g1, g2, g3]
        osems = [o0, o1, o2, o3]

        def gather(g, b, sem):
            pltpu.async_copy(table_hbm.at[idx_v.at[g]], rows_v.at[b], sem)

        def gwait(g, b, sem):
            pltpu.make_async_copy(table_hbm.at[idx_v.at[g]], rows_v.at[b], sem).wait()

        def store(g, b, sem):
            pltpu.async_copy(rows_v.at[b], out_hbm.at[pl.ds(base + g * chunk, chunk)], sem)

        def swait(g, b, sem):
            pltpu.make_async_copy(
                rows_v.at[b], out_hbm.at[pl.ds(base + g * chunk, chunk)], sem).wait()

        for b in range(4):
            gather(b, b, gsems[b])

        def step(s, carry):
            g0_ = 4 * s
            for b in range(4):
                gwait(g0_ + b, b, gsems[b])
                store(g0_ + b, b, osems[b])
            for b in range(4):
                gn = g0_ + 4 + b

                @pl.when(gn < chunks)
                def _():
                    swait(g0_ + b, b, osems[b])
                    gather(gn, b, gsems[b])
            return carry

        lax.fori_loop(0, chunks // 4, step, 0)
        for b in range(4):
            swait(chunks - 4 + b, b, osems[b])

    return k(table, idx3)


def _tc_ffn_paired(epair, W1p, b1p, W2p, b2p, W3pT, b3p):
    m, _ = epair.shape
    blk = 4096
    grid = m // blk

    def body(e_ref, w1_ref, b1_ref, w2_ref, b2_ref, w3t_ref, b3_ref, x_ref):
        e = e_ref[...]
        h = jnp.maximum(jnp.dot(e, w1_ref[...], preferred_element_type=jnp.float32) + b1_ref[...], 0.0)
        h = jnp.maximum(jnp.dot(h, w2_ref[...], preferred_element_type=jnp.float32) + b2_ref[...], 0.0)
        xt = jax.lax.dot_general(w3t_ref[...], h, (((1,), (1,)), ((), ())),
                                 preferred_element_type=jnp.float32)
        x_ref[...] = xt + b3_ref[...]

    return pl.pallas_call(
        body,
        grid=(grid,),
        in_specs=[
            pl.BlockSpec((blk, 128), lambda i: (i, 0)),
            pl.BlockSpec((128, 120), lambda i: (0, 0)),
            pl.BlockSpec((1, 120), lambda i: (0, 0)),
            pl.BlockSpec((120, 120), lambda i: (0, 0)),
            pl.BlockSpec((1, 120), lambda i: (0, 0)),
            pl.BlockSpec((2, 120), lambda i: (0, 0)),
            pl.BlockSpec((2, 1), lambda i: (0, 0)),
        ],
        out_specs=pl.BlockSpec((2, blk), lambda i: (0, i)),
        out_shape=jax.ShapeDtypeStruct((2, m), jnp.float32),
    )(epair, W1p, b1p.reshape(1, 120), W2p, b2p.reshape(1, 120), W3pT, b3p)


def kernel(poi_no, table, W1, b1, W2, b2, W3, b3):
    b, hist = poi_no.shape
    n = b * hist
    d = table.shape[1]
    half = n // 2
    chunks = half // (NW * CHUNK)
    idx_flat = poi_no.astype(jnp.int32).reshape(n)
    idx_a = idx_flat[:half].reshape(NW, chunks, CHUNK)
    idx_b = idx_flat[half:].reshape(NW, chunks, CHUNK)
    e_a = _sc_gather(table, idx_a)
    e_b = _sc_gather(table, idx_b)

    h1 = W1.shape[1]
    h2 = W2.shape[1]
    z12 = jnp.zeros((h1, h2), jnp.float32)
    W1p = jnp.block([[W1, jnp.zeros((d, h1), jnp.float32)],
                     [jnp.zeros((d, h1), jnp.float32), W1]])
    W2p = jnp.block([[W2, z12], [z12, W2]])
    W3pT = jnp.block([[W3.T, jnp.zeros((1, h2), jnp.float32)],
                      [jnp.zeros((1, h2), jnp.float32), W3.T]])
    b1p = jnp.concatenate([b1, b1])
    b2p = jnp.concatenate([b2, b2])
    b3p = jnp.concatenate([b3, b3]).reshape(2, 1)

    x2a = _tc_ffn_paired(e_a.reshape(half // 2, 2 * d), W1p, b1p, W2p, b2p, W3pT, b3p)
    x2b = _tc_ffn_paired(e_b.reshape(half // 2, 2 * d), W1p, b1p, W2p, b2p, W3pT, b3p)
    xa = jnp.stack([x2a[0], x2a[1]], axis=1)
    xb = jnp.stack([x2b[0], x2b[1]], axis=1)
    x_flat = jnp.concatenate([xa.reshape(half), xb.reshape(half)])
    embed = jnp.concatenate([e_a, e_b], axis=0)
    return x_flat.reshape(b, hist, 1), embed.reshape(b, hist, d)


# speedup vs baseline: 1.3932x; 1.3932x over previous
import functools

import jax
import jax.numpy as jnp
from jax import lax
from jax.experimental import pallas as pl
from jax.experimental.pallas import tpu as pltpu
from jax.experimental.pallas import tpu_sc as plsc

NC = 2
NS = 16
NW = NC * NS
CHUNK = 128


def _sc_gather(table, idx3):
    _, chunks, chunk = idx3.shape
    d = table.shape[1]
    per_w = chunks * chunk
    mesh = plsc.VectorSubcoreMesh(core_axis_name="c", subcore_axis_name="s")

    @functools.partial(
        pl.kernel,
        mesh=mesh,
        compiler_params=pltpu.CompilerParams(use_tc_tiling_on_sc=False),
        out_type=jax.ShapeDtypeStruct((NW * per_w, d), jnp.float32),
        scratch_types=(
            [pltpu.VMEM((chunks, chunk), jnp.int32),
             pltpu.VMEM((8, chunk, d), jnp.float32)]
            + [pltpu.SemaphoreType.DMA] * 16
        ),
    )
    def k(table_hbm, idx_hbm, out_hbm, idx_v, rows_v, *sems):
        wid = lax.axis_index("s") * NC + lax.axis_index("c")
        pltpu.sync_copy(idx_hbm.at[wid], idx_v)
        base = wid * per_w
        gsems = sems[:8]
        osems = sems[8:]

        def gather(g, b, sem):
            pltpu.async_copy(table_hbm.at[idx_v.at[g]], rows_v.at[b], sem)

        def gwait(g, b, sem):
            pltpu.make_async_copy(table_hbm.at[idx_v.at[g]], rows_v.at[b], sem).wait()

        def store(g, b, sem):
            pltpu.async_copy(rows_v.at[b], out_hbm.at[pl.ds(base + g * chunk, chunk)], sem)

        def swait(g, b, sem):
            pltpu.make_async_copy(
                rows_v.at[b], out_hbm.at[pl.ds(base + g * chunk, chunk)], sem).wait()

        for b in range(8):
            gather(b, b, gsems[b])

        def step(s, carry):
            g0_ = 8 * s
            for b in range(8):
                gwait(g0_ + b, b, gsems[b])
                store(g0_ + b, b, osems[b])
            for b in range(8):
                gn = g0_ + 8 + b

                @pl.when(gn < chunks)
                def _():
                    swait(g0_ + b, b, osems[b])
                    gather(gn, b, gsems[b])
            return carry

        lax.fori_loop(0, chunks // 8, step, 0)
        for b in range(8):
            swait(chunks - 8 + b, b, osems[b])

    return k(table, idx3)


def _tc_ffn_paired(epair, W1p, b1p, W2p, b2p, W3pT, b3p):
    m, _ = epair.shape
    blk = 4096
    grid = m // blk

    def body(e_ref, w1_ref, b1_ref, w2_ref, b2_ref, w3t_ref, b3_ref, x_ref):
        e = e_ref[...]
        h = jnp.maximum(jnp.dot(e, w1_ref[...], preferred_element_type=jnp.float32) + b1_ref[...], 0.0)
        h = jnp.maximum(jnp.dot(h, w2_ref[...], preferred_element_type=jnp.float32) + b2_ref[...], 0.0)
        xt = jax.lax.dot_general(w3t_ref[...], h, (((1,), (1,)), ((), ())),
                                 preferred_element_type=jnp.float32)
        x_ref[...] = xt + b3_ref[...]

    return pl.pallas_call(
        body,
        grid=(grid,),
        in_specs=[
            pl.BlockSpec((blk, 128), lambda i: (i, 0)),
            pl.BlockSpec((128, 120), lambda i: (0, 0)),
            pl.BlockSpec((1, 120), lambda i: (0, 0)),
            pl.BlockSpec((120, 120), lambda i: (0, 0)),
            pl.BlockSpec((1, 120), lambda i: (0, 0)),
            pl.BlockSpec((2, 120), lambda i: (0, 0)),
            pl.BlockSpec((2, 1), lambda i: (0, 0)),
        ],
        out_specs=pl.BlockSpec((2, blk), lambda i: (0, i)),
        out_shape=jax.ShapeDtypeStruct((2, m), jnp.float32),
    )(epair, W1p, b1p.reshape(1, 120), W2p, b2p.reshape(1, 120), W3pT, b3p)


def kernel(poi_no, table, W1, b1, W2, b2, W3, b3):
    b, hist = poi_no.shape
    n = b * hist
    d = table.shape[1]
    chunks = n // (NW * CHUNK)
    idx3 = poi_no.astype(jnp.int32).reshape(NW, chunks, CHUNK)
    embed_flat = _sc_gather(table, idx3)
    epair = embed_flat.reshape(n // 2, 2 * d)
    h1 = W1.shape[1]
    h2 = W2.shape[1]
    z12 = jnp.zeros((h1, h2), jnp.float32)
    W1p = jnp.block([[W1, jnp.zeros((d, h1), jnp.float32)],
                     [jnp.zeros((d, h1), jnp.float32), W1]])
    W2p = jnp.block([[W2, z12], [z12, W2]])
    W3pT = jnp.block([[W3.T, jnp.zeros((1, h2), jnp.float32)],
                      [jnp.zeros((1, h2), jnp.float32), W3.T]])
    b1p = jnp.concatenate([b1, b1])
    b2p = jnp.concatenate([b2, b2])
    b3p = jnp.concatenate([b3, b3]).reshape(2, 1)
    x2 = _tc_ffn_paired(epair, W1p, b1p, W2p, b2p, W3pT, b3p)
    x_flat = jnp.stack([x2[0], x2[1]], axis=1)
    return x_flat.reshape(b, hist, 1), embed_flat.reshape(b, hist, d)
